# baseline (speedup 1.0000x reference)
import functools

import jax
import jax.numpy as jnp
from jax.experimental import pallas as pl
from jax.experimental.pallas import tpu as pltpu

_LAMBDA1 = 1.0
_LAMBDA2 = 0.5
_TC = 4096
_SUB = 512

_I32_MAX = 2**31 - 1


def _loss_kernel(y1_ref, g_ref, c_ref, out_ref, vals, r0vals, cand, aug_ref, *, m):
    step = pl.program_id(0)
    nsteps = m // _TC
    n, d = y1_ref.shape

    y1 = y1_ref[...]

    @pl.when(step == 0)
    def _init():
        for s in range(2):
            vals[s] = jnp.float32(jnp.inf)
            r0vals[s] = jnp.float32(jnp.inf)
        a2 = jnp.sum(y1 * y1, axis=1, keepdims=True)
        ones = jnp.ones((n, 1), dtype=jnp.float32)
        aug_ref[...] = jnp.concatenate([y1, a2, ones], axis=1)

    yaug = aug_ref[...]

    def process(tab_ref, slot, sub):
        t = tab_ref[pl.ds(sub * _SUB, _SUB), :]
        b2 = jnp.sum(t * t, axis=1, keepdims=True)
        ones = jnp.ones((_SUB, 1), dtype=jnp.float32)
        taug = jnp.concatenate([-2.0 * t, ones, b2], axis=1)
        sqs = jnp.dot(yaug, taug.T, preferred_element_type=jnp.float32)
        tmin = jnp.min(sqs)
        vals[slot] = jnp.minimum(vals[slot], tmin)

        sq0 = sqs[0:1, :]
        r0min = jnp.min(sq0)

        @pl.when(r0min < r0vals[slot])
        def _row0():
            r0vals[slot] = r0min
            cols = jax.lax.broadcasted_iota(jnp.int32, (1, _SUB), 1)
            cid = jnp.min(jnp.where(sq0 == r0min, cols, jnp.int32(_I32_MAX)))
            base = 2 * slot
            cand[base : base + 1, :] = tab_ref[pl.ds(sub * _SUB + cid, 1), :]

        @pl.when(step == nsteps - 1)
        def _last_row():
            base = 2 * slot
            cand[base + 1 : base + 2, :] = tab_ref[_TC - 1 : _TC, :]

    for sub in range(_TC // _SUB):
        process(g_ref, 0, sub)
        process(c_ref, 1, sub)

    @pl.when(step == nsteps - 1)
    def _finalize():
        def point(slot):
            base = 2 * slot
            return jnp.where(
                r0vals[slot] <= vals[slot],
                cand[base : base + 1, :],
                cand[base + 1 : base + 2, :],
            )

        gp = point(0)
        cp = point(1)
        glue = jnp.mean((gp - y1) ** 2)
        cut = jnp.mean((cp - y1) ** 2)
        out_ref[0, 0] = _LAMBDA1 * glue - _LAMBDA2 * cut


def kernel(y1, gluers, cutters):
    n, d = y1.shape
    m = gluers.shape[0]
    nsteps = m // _TC
    out = pl.pallas_call(
        functools.partial(_loss_kernel, m=m),
        grid=(nsteps,),
        in_specs=[
            pl.BlockSpec((n, d), lambda c: (0, 0)),
            pl.BlockSpec((_TC, d), lambda c: (c, 0)),
            pl.BlockSpec((_TC, d), lambda c: (c, 0)),
        ],
        out_specs=pl.BlockSpec((1, 1), lambda c: (0, 0), memory_space=pltpu.SMEM),
        out_shape=jax.ShapeDtypeStruct((1, 1), jnp.float32),
        scratch_shapes=[
            pltpu.SMEM((2,), jnp.float32),
            pltpu.SMEM((2,), jnp.float32),
            pltpu.VMEM((4, d), jnp.float32),
            pltpu.VMEM((n, d + 2), jnp.float32),
        ],
    )(y1, gluers, cutters)
    return out[0, 0]


# speedup vs baseline: 1.0133x; 1.0133x over previous
import functools

import jax
import jax.numpy as jnp
from jax.experimental import pallas as pl
from jax.experimental.pallas import tpu as pltpu

_LAMBDA1 = 1.0
_LAMBDA2 = 0.5
_TC = 2048
_SUB = 512

_I32_MAX = 2**31 - 1


def _loss_kernel(y1_ref, g_ref, c_ref, out_ref, vals, r0vals, cand, aug_ref, *, m):
    step = pl.program_id(0)
    nsteps = m // _TC
    n, d = y1_ref.shape

    y1 = y1_ref[...]

    @pl.when(step == 0)
    def _init():
        for s in range(2):
            vals[s] = jnp.float32(jnp.inf)
            r0vals[s] = jnp.float32(jnp.inf)
        aug_ref[...] = jnp.sum(y1 * y1, axis=1, keepdims=True)

    a2 = aug_ref[...]

    def process(tab_ref, slot, sub):
        t = tab_ref[pl.ds(sub * _SUB, _SUB), :]
        b2 = jnp.sum(t * t, axis=1)
        sqs = (
            jnp.dot(y1, -2.0 * t.T, preferred_element_type=jnp.float32)
            + b2[None, :]
            + a2
        )
        tmin = jnp.min(sqs)
        vals[slot] = jnp.minimum(vals[slot], tmin)

        sq0 = sqs[0:1, :]
        r0min = jnp.min(sq0)

        @pl.when(r0min < r0vals[slot])
        def _row0():
            r0vals[slot] = r0min
            cols = jax.lax.broadcasted_iota(jnp.int32, (1, _SUB), 1)
            cid = jnp.min(jnp.where(sq0 == r0min, cols, jnp.int32(_I32_MAX)))
            base = 2 * slot
            cand[base : base + 1, :] = tab_ref[pl.ds(sub * _SUB + cid, 1), :]

        @pl.when(step == nsteps - 1)
        def _last_row():
            base = 2 * slot
            cand[base + 1 : base + 2, :] = tab_ref[_TC - 1 : _TC, :]

    for sub in range(_TC // _SUB):
        process(g_ref, 0, sub)
        process(c_ref, 1, sub)

    @pl.when(step == nsteps - 1)
    def _finalize():
        def point(slot):
            base = 2 * slot
            return jnp.where(
                r0vals[slot] <= vals[slot],
                cand[base : base + 1, :],
                cand[base + 1 : base + 2, :],
            )

        gp = point(0)
        cp = point(1)
        glue = jnp.mean((gp - y1) ** 2)
        cut = jnp.mean((cp - y1) ** 2)
        out_ref[0, 0] = _LAMBDA1 * glue - _LAMBDA2 * cut


def kernel(y1, gluers, cutters):
    n, d = y1.shape
    m = gluers.shape[0]
    nsteps = m // _TC
    out = pl.pallas_call(
        functools.partial(_loss_kernel, m=m),
        grid=(nsteps,),
        in_specs=[
            pl.BlockSpec((n, d), lambda c: (0, 0)),
            pl.BlockSpec((_TC, d), lambda c: (c, 0)),
            pl.BlockSpec((_TC, d), lambda c: (c, 0)),
        ],
        out_specs=pl.BlockSpec((1, 1), lambda c: (0, 0), memory_space=pltpu.SMEM),
        out_shape=jax.ShapeDtypeStruct((1, 1), jnp.float32),
        scratch_shapes=[
            pltpu.SMEM((2,), jnp.float32),
            pltpu.SMEM((2,), jnp.float32),
            pltpu.VMEM((4, d), jnp.float32),
            pltpu.VMEM((n, 1), jnp.float32),
        ],
    )(y1, gluers, cutters)
    return out[0, 0]


# speedup vs baseline: 1.0592x; 1.0453x over previous
import functools

import jax
import jax.numpy as jnp
from jax.experimental import pallas as pl
from jax.experimental.pallas import tpu as pltpu

_LAMBDA1 = 1.0
_LAMBDA2 = 0.5
_TC = 2048
_SUB = 512

_I32_MAX = 2**31 - 1


def _loss_kernel(y1_ref, g_ref, c_ref, out_ref, vals, r0vals, cand, aug_ref, *, m):
    step = pl.program_id(0)
    nsteps = m // _TC
    n, d = y1_ref.shape

    y1 = y1_ref[...]

    @pl.when(step == 0)
    def _init():
        for s in range(2):
            vals[s] = jnp.float32(jnp.inf)
            r0vals[s] = jnp.float32(jnp.inf)
        a2 = jnp.sum(y1 * y1, axis=1, keepdims=True)
        ones = jnp.ones((n, 1), dtype=jnp.float32)
        aug_ref[...] = jnp.concatenate([y1, a2, ones], axis=1)

    yaug = aug_ref[...]

    def dist(tab_ref, sub):
        t = tab_ref[pl.ds(sub * _SUB, _SUB), :]
        b2 = jnp.sum(t * t, axis=1, keepdims=True)
        ones = jnp.ones((_SUB, 1), dtype=jnp.float32)
        taug = jnp.concatenate([-2.0 * t, ones, b2], axis=1)
        return jnp.dot(yaug, taug.T, preferred_element_type=jnp.float32)

    def reduce(tab_ref, slot, sub, sqs):
        tmin = jnp.min(sqs)
        vals[slot] = jnp.minimum(vals[slot], tmin)

        sq0 = sqs[0:1, :]
        r0min = jnp.min(sq0)

        @pl.when(r0min < r0vals[slot])
        def _row0():
            r0vals[slot] = r0min
            cols = jax.lax.broadcasted_iota(jnp.int32, (1, _SUB), 1)
            cid = jnp.min(jnp.where(sq0 == r0min, cols, jnp.int32(_I32_MAX)))
            base = 2 * slot
            cand[base : base + 1, :] = tab_ref[pl.ds(sub * _SUB + cid, 1), :]

        @pl.when(step == nsteps - 1)
        def _last_row():
            base = 2 * slot
            cand[base + 1 : base + 2, :] = tab_ref[_TC - 1 : _TC, :]

    work = [
        (ref, slot, sub)
        for sub in range(_TC // _SUB)
        for ref, slot in ((g_ref, 0), (c_ref, 1))
    ]
    prev = None
    for ref, slot, sub in work:
        sqs = dist(ref, sub)
        if prev is not None:
            reduce(*prev)
        prev = (ref, slot, sub, sqs)
    reduce(*prev)

    @pl.when(step == nsteps - 1)
    def _finalize():
        def point(slot):
            base = 2 * slot
            return jnp.where(
                r0vals[slot] <= vals[slot],
                cand[base : base + 1, :],
                cand[base + 1 : base + 2, :],
            )

        gp = point(0)
        cp = point(1)
        glue = jnp.mean((gp - y1) ** 2)
        cut = jnp.mean((cp - y1) ** 2)
        out_ref[0, 0] = _LAMBDA1 * glue - _LAMBDA2 * cut


def kernel(y1, gluers, cutters):
    n, d = y1.shape
    m = gluers.shape[0]
    nsteps = m // _TC
    out = pl.pallas_call(
        functools.partial(_loss_kernel, m=m),
        grid=(nsteps,),
        in_specs=[
            pl.BlockSpec((n, d), lambda c: (0, 0)),
            pl.BlockSpec((_TC, d), lambda c: (c, 0)),
            pl.BlockSpec((_TC, d), lambda c: (c, 0)),
        ],
        out_specs=pl.BlockSpec((1, 1), lambda c: (0, 0), memory_space=pltpu.SMEM),
        out_shape=jax.ShapeDtypeStruct((1, 1), jnp.float32),
        scratch_shapes=[
            pltpu.SMEM((2,), jnp.float32),
            pltpu.SMEM((2,), jnp.float32),
            pltpu.VMEM((4, d), jnp.float32),
            pltpu.VMEM((n, d + 2), jnp.float32),
        ],
    )(y1, gluers, cutters)
    return out[0, 0]
